# baseline (speedup 1.0000x reference)
import functools

import jax
import jax.numpy as jnp
from jax import lax
from jax.experimental import pallas as pl
from jax.experimental.pallas import tpu as pltpu
from jax.experimental.pallas import tpu_sc as plsc

_BATCH = 16384
_EMB = 32
_VROW = 128
_VR_PER = _VROW // _EMB
_NC = 2
_NS = 16
_NW = _NC * _NS
_BPW = _BATCH // _NW
_CHUNK = 128
_K = _BPW // _CHUNK


def _make_sc_gather():
    mesh = plsc.VectorSubcoreMesh(core_axis_name="c", subcore_axis_name="s")

    @functools.partial(
        pl.kernel,
        mesh=mesh,
        out_type=(
            jax.ShapeDtypeStruct((_BATCH, _VROW), jnp.float32),
            jax.ShapeDtypeStruct((_BATCH, _VROW), jnp.float32),
        ),
        scratch_types=[
            pltpu.VMEM((_K, _CHUNK), jnp.int32),
            pltpu.VMEM((_K, _CHUNK), jnp.int32),
            pltpu.VMEM((_CHUNK, _VROW), jnp.float32),
            pltpu.VMEM((_CHUNK, _VROW), jnp.float32),
            pltpu.SemaphoreType.DMA,
            pltpu.SemaphoreType.DMA,
        ],
    )
    def gather(uv_hbm, iv_hbm, tu_hbm, ti_hbm, xu_out, xi_out,
               uidx, iidx, ubuf, ibuf, usem, isem):
        wid = lax.axis_index("s") * _NC + lax.axis_index("c")
        base = wid * _BPW
        pltpu.sync_copy(uv_hbm.at[wid], uidx)
        pltpu.sync_copy(iv_hbm.at[wid], iidx)
        for j in range(_K):
            cu = pltpu.async_copy(tu_hbm.at[uidx.at[j]], ubuf, usem)
            ci = pltpu.async_copy(ti_hbm.at[iidx.at[j]], ibuf, isem)
            cu.wait()
            pltpu.sync_copy(ubuf, xu_out.at[pl.ds(base + j * _CHUNK, _CHUNK)])
            ci.wait()
            pltpu.sync_copy(ibuf, xi_out.at[pl.ds(base + j * _CHUNK, _CHUNK)])

    return gather


def _mlp_body(xu, xi, su, si, w1a, w1b, b1, w2, b2, w3, b3, wl, bl, out):
    def extract(x128, s):
        e = x128[:, 0:_EMB]
        for k in range(1, _VR_PER):
            e = jnp.where(s == k, x128[:, k * _EMB:(k + 1) * _EMB], e)
        return e

    ue = extract(xu[...], su[...])
    ie = extract(xi[...], si[...])
    h = jnp.dot(ue, w1a[...], preferred_element_type=jnp.float32)
    h = h + jnp.dot(ie, w1b[...], preferred_element_type=jnp.float32)
    h = jnp.maximum(h + b1[...], 0.0)
    h = jnp.maximum(
        jnp.dot(h, w2[...], preferred_element_type=jnp.float32) + b2[...], 0.0)
    h = jnp.maximum(
        jnp.dot(h, w3[...], preferred_element_type=jnp.float32) + b3[...], 0.0)
    logit = jnp.dot(h, wl[...], preferred_element_type=jnp.float32) + bl[...]
    out[...] = 1.0 / (1.0 + jnp.exp(-logit))


_BB = 2048


def _mlp(xu, xi, su, si, w1a, w1b, b1, w2, b2, w3, b3, wl, bl):
    grid = (_BATCH // _BB,)
    brow = lambda w: pl.BlockSpec((_BB, w), lambda i: (i, 0))
    bfull = lambda a: pl.BlockSpec(a.shape, lambda i: tuple(0 for _ in a.shape))
    return pl.pallas_call(
        _mlp_body,
        grid=grid,
        in_specs=[
            brow(_VROW), brow(_VROW), brow(1), brow(1),
            bfull(w1a), bfull(w1b), bfull(b1),
            bfull(w2), bfull(b2),
            bfull(w3), bfull(b3),
            bfull(wl), bfull(bl),
        ],
        out_specs=pl.BlockSpec((_BB, 1), lambda i: (i, 0)),
        out_shape=jax.ShapeDtypeStruct((_BATCH, 1), jnp.float32),
    )(xu, xi, su, si, w1a, w1b, b1, w2, b2, w3, b3, wl, bl)


def kernel(users, items, user_emb, movie_emb, W1, b1, W2, b2, W3, b3, Wl, bl):
    u = users.astype(jnp.int32)
    i = items.astype(jnp.int32)
    uv = (u >> 2).reshape(_NW, _K, _CHUNK)
    iv = (i >> 2).reshape(_NW, _K, _CHUNK)
    su = (u & (_VR_PER - 1)).reshape(_BATCH, 1)
    si = (i & (_VR_PER - 1)).reshape(_BATCH, 1)
    tu = user_emb.reshape(-1, _VROW)
    ti = movie_emb.reshape(-1, _VROW)
    xu, xi = _make_sc_gather()(uv, iv, tu, ti)
    return _mlp(
        xu, xi, su, si,
        W1[:_EMB], W1[_EMB:], b1.reshape(1, -1),
        W2, b2.reshape(1, -1),
        W3, b3.reshape(1, -1),
        Wl, bl.reshape(1, 1),
    )


# speedup vs baseline: 1.5354x; 1.5354x over previous
import functools

import jax
import jax.numpy as jnp
from jax import lax
from jax.experimental import pallas as pl
from jax.experimental.pallas import tpu as pltpu
from jax.experimental.pallas import tpu_sc as plsc

_BATCH = 16384
_EMB = 32
_NC = 2
_NS = 16
_NW = _NC * _NS
_BPW = _BATCH // _NW
_NSEM = 8


def _make_sc_gather():
    mesh = plsc.VectorSubcoreMesh(core_axis_name="c", subcore_axis_name="s")

    @functools.partial(
        pl.kernel,
        mesh=mesh,
        out_type=(
            jax.ShapeDtypeStruct((_BATCH, _EMB), jnp.float32),
            jax.ShapeDtypeStruct((_BATCH, _EMB), jnp.float32),
        ),
        scratch_types=[
            pltpu.VMEM((_BPW,), jnp.int32),
            pltpu.VMEM((_BPW,), jnp.int32),
            pltpu.VMEM((_BPW // 2, _EMB), jnp.float32),
            pltpu.VMEM((_BPW // 2, _EMB), jnp.float32),
            [pltpu.SemaphoreType.DMA] * _NSEM,
            pltpu.SemaphoreType.DMA,
        ],
        compiler_params=pltpu.CompilerParams(use_tc_tiling_on_sc=True),
    )
    def gather(users_hbm, items_hbm, tu_hbm, ti_hbm, ue_out, ie_out,
               uidx, iidx, ubuf, ibuf, sems, osem):
        wid = lax.axis_index("s") * _NC + lax.axis_index("c")
        base = wid * _BPW
        pltpu.sync_copy(users_hbm.at[pl.ds(base, _BPW)], uidx)
        pltpu.sync_copy(items_hbm.at[pl.ds(base, _BPW)], iidx)

        half = _BPW // 2
        for h in range(2):
            def group(g, _):
                uv = uidx[pl.ds(h * half + g * 16, 16)]
                iv = iidx[pl.ds(h * half + g * 16, 16)]
                for l in range(16):
                    pltpu.async_copy(tu_hbm.at[uv[l]], ubuf.at[g * 16 + l], sems[0])
                    pltpu.async_copy(ti_hbm.at[iv[l]], ibuf.at[g * 16 + l], sems[1])
                return ()

            lax.fori_loop(0, half // 16, group, ())
            pltpu.make_async_copy(tu_hbm.at[pl.ds(0, half)], ubuf, sems[0]).wait()
            pltpu.make_async_copy(ti_hbm.at[pl.ds(0, half)], ibuf, sems[1]).wait()
            pltpu.sync_copy(ubuf, ue_out.at[pl.ds(base + h * half, half)])
            pltpu.sync_copy(ibuf, ie_out.at[pl.ds(base + h * half, half)])

    return gather


def _mlp_body(ue, ie, w1a, w1b, b1, w2, b2, w3, b3, wl, bl, out):
    h = jnp.dot(ue[...], w1a[...], preferred_element_type=jnp.float32)
    h = h + jnp.dot(ie[...], w1b[...], preferred_element_type=jnp.float32)
    h = jnp.maximum(h + b1[...], 0.0)
    h = jnp.maximum(
        jnp.dot(h, w2[...], preferred_element_type=jnp.float32) + b2[...], 0.0)
    h = jnp.maximum(
        jnp.dot(h, w3[...], preferred_element_type=jnp.float32) + b3[...], 0.0)
    logit = jnp.dot(h, wl[...], preferred_element_type=jnp.float32) + bl[...]
    out[...] = 1.0 / (1.0 + jnp.exp(-logit))


_BB = 2048


def _mlp(ue, ie, w1a, w1b, b1, w2, b2, w3, b3, wl, bl):
    grid = (_BATCH // _BB,)
    brow = lambda w: pl.BlockSpec((_BB, w), lambda i: (i, 0))
    bfull = lambda a: pl.BlockSpec(a.shape, lambda i: tuple(0 for _ in a.shape))
    return pl.pallas_call(
        _mlp_body,
        grid=grid,
        in_specs=[
            brow(_EMB), brow(_EMB),
            bfull(w1a), bfull(w1b), bfull(b1),
            bfull(w2), bfull(b2),
            bfull(w3), bfull(b3),
            bfull(wl), bfull(bl),
        ],
        out_specs=pl.BlockSpec((_BB, 1), lambda i: (i, 0)),
        out_shape=jax.ShapeDtypeStruct((_BATCH, 1), jnp.float32),
    )(ue, ie, w1a, w1b, b1, w2, b2, w3, b3, wl, bl)


def kernel(users, items, user_emb, movie_emb, W1, b1, W2, b2, W3, b3, Wl, bl):
    u = users.astype(jnp.int32)
    i = items.astype(jnp.int32)
    ue, ie = _make_sc_gather()(u, i, user_emb, movie_emb)
    return _mlp(
        ue, ie,
        W1[:_EMB], W1[_EMB:], b1.reshape(1, -1),
        W2, b2.reshape(1, -1),
        W3, b3.reshape(1, -1),
        Wl, bl.reshape(1, 1),
    )
